# baseline (speedup 1.0000x reference)
import functools

import jax
import jax.numpy as jnp
from jax import lax
from jax.experimental import pallas as pl
from jax.experimental.pallas import tpu as pltpu
import jax.experimental.pallas.tpu_sc as plsc

N = 10000
E = 320000
D = 128
H = 128
C = 70

NC = 2
NS = 16
NW = NC * NS
EPW = E // NW
B = 125
NB = EPW // B
CH = NB // 2
NP = 10240
RPT = NP // NS

_mesh = plsc.VectorSubcoreMesh(core_axis_name="c", subcore_axis_name="s",
                               num_cores=NC, num_subcores=NS)


@functools.partial(
    pl.kernel,
    out_type=jax.ShapeDtypeStruct((NC, NP), jnp.float32),
    mesh=_mesh,
    scratch_types=[
        pltpu.VMEM((NB, B), jnp.int32),
        pltpu.VMEM((128,), jnp.float32),
        pltpu.VMEM((RPT,), jnp.float32),
        pltpu.VMEM_SHARED((NP,), jnp.float32),
    ],
)
def _deg_kernel(e4_hbm, out_hbm, dst_v, ones_v, zb_v, deg_sh):
    c = lax.axis_index("c")
    s = lax.axis_index("s")
    wid = c * NS + s

    def fill_ones(i, carry):
        ones_v[pl.ds(i * 16, 16)] = jnp.full((16,), 1.0, jnp.float32)
        return carry
    lax.fori_loop(0, 8, fill_ones, 0)

    def fill_zero(i, carry):
        zb_v[pl.ds(i * 16, 16)] = jnp.zeros((16,), jnp.float32)
        return carry
    lax.fori_loop(0, RPT // 16, fill_zero, 0)

    pltpu.sync_copy(zb_v, deg_sh.at[pl.ds(s * RPT, RPT)])
    pltpu.sync_copy(e4_hbm.at[1].at[wid], dst_v)
    plsc.subcore_barrier()

    def body(j, carry):
        pltpu.sync_copy(ones_v.at[pl.ds(0, B)],
                        deg_sh.at[dst_v.at[j]], add=True)
        return carry
    lax.fori_loop(0, NB, body, 0)

    plsc.subcore_barrier()
    pltpu.sync_copy(deg_sh.at[pl.ds(s * RPT, RPT)],
                    out_hbm.at[c].at[pl.ds(s * RPT, RPT)])


@functools.partial(
    pl.kernel,
    out_type=jax.ShapeDtypeStruct((NC, NP, D), jnp.float32),
    mesh=_mesh,
    scratch_types=[
        pltpu.VMEM((CH, B), jnp.int32),
        pltpu.VMEM((CH, B), jnp.int32),
        pltpu.VMEM((B, D), jnp.float32),
        pltpu.VMEM((B, D), jnp.float32),
        pltpu.VMEM_SHARED((NP, D), jnp.float32),
        pltpu.SemaphoreType.DMA,
        pltpu.SemaphoreType.DMA,
    ],
)
def _agg_kernel(z_hbm, e4_hbm, out_hbm,
                src_v, dst_v, buf_a, buf_b, acc_sh, sem_a, sem_b):
    c = lax.axis_index("c")
    s = lax.axis_index("s")
    wid = c * NS + s

    def zrow(i, carry):
        buf_a[i // 8, pl.ds((i % 8) * 16, 16)] = jnp.zeros((16,), jnp.float32)
        return carry
    lax.fori_loop(0, B * (D // 16), zrow, 0)
    for k in range(RPT // B):
        pltpu.sync_copy(buf_a, acc_sh.at[pl.ds(s * RPT + k * B, B)])
    if RPT % B:
        pltpu.sync_copy(buf_a.at[pl.ds(0, RPT % B)],
                        acc_sh.at[pl.ds(s * RPT + (RPT // B) * B, RPT % B)])

    def load_chunk(ci):
        pltpu.sync_copy(e4_hbm.at[0].at[wid].at[pl.ds(ci * CH, CH)], src_v)
        pltpu.sync_copy(e4_hbm.at[1].at[wid].at[pl.ds(ci * CH, CH)], dst_v)

    load_chunk(0)
    plsc.subcore_barrier()

    def g_start(j, buf, sem):
        pltpu.async_copy(z_hbm.at[src_v.at[j]], buf, sem)

    def g_wait(j, buf, sem):
        pltpu.make_async_copy(z_hbm.at[src_v.at[j]], buf, sem).wait()

    def scat(j, buf):
        pltpu.sync_copy(buf, acc_sh.at[dst_v.at[j]], add=True)

    def run_pipeline(count):
        g_start(0, buf_a, sem_a)
        nloop = (count - 1) // 2 if count % 2 else (count - 2) // 2

        def body(jj, carry):
            j0 = jj * 2
            g_start(j0 + 1, buf_b, sem_b)
            g_wait(j0, buf_a, sem_a)
            scat(j0, buf_a)
            g_start(j0 + 2, buf_a, sem_a)
            g_wait(j0 + 1, buf_b, sem_b)
            scat(j0 + 1, buf_b)
            return carry
        lax.fori_loop(0, nloop, body, 0)

        if count % 2:
            g_wait(count - 1, buf_a, sem_a)
            scat(count - 1, buf_a)
        else:
            g_start(count - 1, buf_b, sem_b)
            g_wait(count - 2, buf_a, sem_a)
            scat(count - 2, buf_a)
            g_wait(count - 1, buf_b, sem_b)
            scat(count - 1, buf_b)

    run_pipeline(CH)
    load_chunk(1)
    run_pipeline(NB - CH)

    plsc.subcore_barrier()
    pltpu.sync_copy(acc_sh.at[pl.ds(s * RPT, RPT)],
                    out_hbm.at[c].at[pl.ds(s * RPT, RPT)])


_BR = 512
_GRID = NP // _BR


def _tc1_body(x_ref, w1_ref, dp_ref, z1_ref, invd_ref):
    invd_row = lax.rsqrt(dp_ref[0:1] + dp_ref[1:2] + 1.0)
    invd_col = invd_row.reshape(_BR, 1)
    y = jnp.dot(x_ref[...], w1_ref[...], preferred_element_type=jnp.float32)
    z1_ref[...] = y * invd_col
    invd_ref[...] = invd_row


def _tc2_body(p_ref, invd_ref, b1_ref, w2_ref, z2_ref):
    invd_col = invd_ref[...].reshape(_BR, 1)
    h1 = jnp.maximum((p_ref[0] + p_ref[1]) * invd_col + b1_ref[...], 0.0)
    z2_ref[...] = jnp.dot(h1, w2_ref[...], preferred_element_type=jnp.float32) * invd_col


def _tc3_body(p_ref, invd_ref, b2_ref, wc_ref, bc_ref, out_ref):
    invd_col = invd_ref[...].reshape(_BR, 1)
    h2 = jnp.maximum((p_ref[0] + p_ref[1]) * invd_col + b2_ref[...], 0.0)
    out_ref[...] = jnp.dot(h2, wc_ref[...], preferred_element_type=jnp.float32) + bc_ref[...]


_tc1 = pl.pallas_call(
    _tc1_body,
    grid=(_GRID,),
    in_specs=[
        pl.BlockSpec((_BR, D), lambda i: (i, 0)),
        pl.BlockSpec((D, H), lambda i: (0, 0)),
        pl.BlockSpec((NC, _BR), lambda i: (0, i)),
    ],
    out_specs=[
        pl.BlockSpec((_BR, H), lambda i: (i, 0)),
        pl.BlockSpec((1, _BR), lambda i: (0, i)),
    ],
    out_shape=[
        jax.ShapeDtypeStruct((NP, H), jnp.float32),
        jax.ShapeDtypeStruct((1, NP), jnp.float32),
    ],
)

_tc2 = pl.pallas_call(
    _tc2_body,
    grid=(_GRID,),
    in_specs=[
        pl.BlockSpec((NC, _BR, H), lambda i: (0, i, 0)),
        pl.BlockSpec((1, _BR), lambda i: (0, i)),
        pl.BlockSpec((1, H), lambda i: (0, 0)),
        pl.BlockSpec((H, H), lambda i: (0, 0)),
    ],
    out_specs=pl.BlockSpec((_BR, H), lambda i: (i, 0)),
    out_shape=jax.ShapeDtypeStruct((NP, H), jnp.float32),
)

_tc3 = pl.pallas_call(
    _tc3_body,
    grid=(_GRID,),
    in_specs=[
        pl.BlockSpec((NC, _BR, H), lambda i: (0, i, 0)),
        pl.BlockSpec((1, _BR), lambda i: (0, i)),
        pl.BlockSpec((1, H), lambda i: (0, 0)),
        pl.BlockSpec((H, C), lambda i: (0, 0)),
        pl.BlockSpec((1, C), lambda i: (0, 0)),
    ],
    out_specs=pl.BlockSpec((_BR, C), lambda i: (i, 0)),
    out_shape=jax.ShapeDtypeStruct((N, C), jnp.float32),
)


def kernel(x, edge_index, W1, b1, W2, b2, Wc, bc):
    e4 = edge_index.reshape(2, NW, NB, B)

    degp = _deg_kernel(e4)
    z1, invd = _tc1(x, W1, degp)
    p1 = _agg_kernel(z1, e4)
    z2 = _tc2(p1, invd, b1.reshape(1, H), W2)
    p2 = _agg_kernel(z2, e4)
    return _tc3(p2, invd, b2.reshape(1, H), Wc, bc.reshape(1, C))


# speedup vs baseline: 1.0824x; 1.0824x over previous
import functools

import jax
import jax.numpy as jnp
from jax import lax
from jax.experimental import pallas as pl
from jax.experimental.pallas import tpu as pltpu
import jax.experimental.pallas.tpu_sc as plsc

N = 10000
E = 320000
D = 128
H = 128
C = 70

NC = 2
NS = 16
NW = NC * NS
EPW = E // NW
B = 125
NB = EPW // B
CH = NB // 2
NP = 10240
RPT = NP // NS

_mesh = plsc.VectorSubcoreMesh(core_axis_name="c", subcore_axis_name="s",
                               num_cores=NC, num_subcores=NS)


@functools.partial(
    pl.kernel,
    out_type=jax.ShapeDtypeStruct((NC, NP), jnp.float32),
    mesh=_mesh,
    scratch_types=[
        pltpu.VMEM((NB, B), jnp.int32),
        pltpu.VMEM((128,), jnp.float32),
        pltpu.VMEM((RPT,), jnp.float32),
        pltpu.VMEM_SHARED((NP,), jnp.float32),
    ],
)
def _deg_kernel(e4_hbm, out_hbm, dst_v, ones_v, zb_v, deg_sh):
    c = lax.axis_index("c")
    s = lax.axis_index("s")
    wid = c * NS + s

    def fill_ones(i, carry):
        ones_v[pl.ds(i * 16, 16)] = jnp.full((16,), 1.0, jnp.float32)
        return carry
    lax.fori_loop(0, 8, fill_ones, 0)

    def fill_zero(i, carry):
        zb_v[pl.ds(i * 16, 16)] = jnp.zeros((16,), jnp.float32)
        return carry
    lax.fori_loop(0, RPT // 16, fill_zero, 0)

    pltpu.sync_copy(zb_v, deg_sh.at[pl.ds(s * RPT, RPT)])
    pltpu.sync_copy(e4_hbm.at[1].at[wid], dst_v)
    plsc.subcore_barrier()

    def body(j, carry):
        pltpu.sync_copy(ones_v.at[pl.ds(0, B)],
                        deg_sh.at[dst_v.at[j]], add=True)
        return carry
    lax.fori_loop(0, NB, body, 0)

    plsc.subcore_barrier()
    pltpu.sync_copy(deg_sh.at[pl.ds(s * RPT, RPT)],
                    out_hbm.at[c].at[pl.ds(s * RPT, RPT)])


@functools.partial(
    pl.kernel,
    out_type=jax.ShapeDtypeStruct((NC, NP, D), jnp.float32),
    mesh=_mesh,
    scratch_types=[
        pltpu.VMEM((CH, B), jnp.int32),
        pltpu.VMEM((CH, B), jnp.int32),
        pltpu.VMEM((B, D), jnp.float32),
        pltpu.VMEM((B, D), jnp.float32),
        pltpu.VMEM_SHARED((NP, D), jnp.float32),
        pltpu.SemaphoreType.DMA,
        pltpu.SemaphoreType.DMA,
    ],
)
def _agg_kernel(z_hbm, e4_hbm, out_hbm,
                src_v, dst_v, buf_a, buf_b, acc_sh, sem_a, sem_b):
    c = lax.axis_index("c")
    s = lax.axis_index("s")
    wid = c * NS + s

    def zrow(i, carry):
        buf_a[i // 8, pl.ds((i % 8) * 16, 16)] = jnp.zeros((16,), jnp.float32)
        return carry
    lax.fori_loop(0, B * (D // 16), zrow, 0)
    for k in range(RPT // B):
        pltpu.sync_copy(buf_a, acc_sh.at[pl.ds(s * RPT + k * B, B)])
    if RPT % B:
        pltpu.sync_copy(buf_a.at[pl.ds(0, RPT % B)],
                        acc_sh.at[pl.ds(s * RPT + (RPT // B) * B, RPT % B)])

    def load_chunk(ci):
        pltpu.sync_copy(e4_hbm.at[0].at[wid].at[pl.ds(ci * CH, CH)], src_v)
        pltpu.sync_copy(e4_hbm.at[1].at[wid].at[pl.ds(ci * CH, CH)], dst_v)

    load_chunk(0)
    plsc.subcore_barrier()

    def g_start(j, buf, sem):
        pltpu.async_copy(z_hbm.at[src_v.at[j]], buf, sem)

    def g_wait(j, buf, sem):
        pltpu.make_async_copy(z_hbm.at[src_v.at[j]], buf, sem).wait()

    def scat(j, buf):
        pltpu.sync_copy(buf, acc_sh.at[dst_v.at[j]], add=True)

    def run_pipeline(count):
        g_start(0, buf_a, sem_a)
        nloop = (count - 1) // 2 if count % 2 else (count - 2) // 2

        def body(jj, carry):
            j0 = jj * 2
            g_start(j0 + 1, buf_b, sem_b)
            g_wait(j0, buf_a, sem_a)
            scat(j0, buf_a)
            g_start(j0 + 2, buf_a, sem_a)
            g_wait(j0 + 1, buf_b, sem_b)
            scat(j0 + 1, buf_b)
            return carry
        lax.fori_loop(0, nloop, body, 0)

        if count % 2:
            g_wait(count - 1, buf_a, sem_a)
            scat(count - 1, buf_a)
        else:
            g_start(count - 1, buf_b, sem_b)
            g_wait(count - 2, buf_a, sem_a)
            scat(count - 2, buf_a)
            g_wait(count - 1, buf_b, sem_b)
            scat(count - 1, buf_b)

    run_pipeline(CH)
    load_chunk(1)
    run_pipeline(NB - CH)

    plsc.subcore_barrier()
    pltpu.sync_copy(acc_sh.at[pl.ds(s * RPT, RPT)],
                    out_hbm.at[c].at[pl.ds(s * RPT, RPT)])


_BR = 2048
_GRID = NP // _BR


def _tc1_body(x_ref, w1_ref, dp_ref, z1_ref, invd_ref):
    invd_row = lax.rsqrt(dp_ref[0:1] + dp_ref[1:2] + 1.0)
    invd_col = invd_row.reshape(_BR, 1)
    y = jnp.dot(x_ref[...], w1_ref[...], preferred_element_type=jnp.float32)
    z1_ref[...] = y * invd_col
    invd_ref[...] = invd_row


def _tc2_body(p_ref, invd_ref, b1_ref, w2_ref, z2_ref):
    invd_col = invd_ref[...].reshape(_BR, 1)
    h1 = jnp.maximum((p_ref[0] + p_ref[1]) * invd_col + b1_ref[...], 0.0)
    z2_ref[...] = jnp.dot(h1, w2_ref[...], preferred_element_type=jnp.float32) * invd_col


def _tc3_body(p_ref, invd_ref, b2_ref, wc_ref, bc_ref, out_ref):
    invd_col = invd_ref[...].reshape(_BR, 1)
    h2 = jnp.maximum((p_ref[0] + p_ref[1]) * invd_col + b2_ref[...], 0.0)
    out_ref[...] = jnp.dot(h2, wc_ref[...], preferred_element_type=jnp.float32) + bc_ref[...]


_tc1 = pl.pallas_call(
    _tc1_body,
    grid=(_GRID,),
    in_specs=[
        pl.BlockSpec((_BR, D), lambda i: (i, 0)),
        pl.BlockSpec((D, H), lambda i: (0, 0)),
        pl.BlockSpec((NC, _BR), lambda i: (0, i)),
    ],
    out_specs=[
        pl.BlockSpec((_BR, H), lambda i: (i, 0)),
        pl.BlockSpec((1, _BR), lambda i: (0, i)),
    ],
    out_shape=[
        jax.ShapeDtypeStruct((NP, H), jnp.float32),
        jax.ShapeDtypeStruct((1, NP), jnp.float32),
    ],
)

_tc2 = pl.pallas_call(
    _tc2_body,
    grid=(_GRID,),
    in_specs=[
        pl.BlockSpec((NC, _BR, H), lambda i: (0, i, 0)),
        pl.BlockSpec((1, _BR), lambda i: (0, i)),
        pl.BlockSpec((1, H), lambda i: (0, 0)),
        pl.BlockSpec((H, H), lambda i: (0, 0)),
    ],
    out_specs=pl.BlockSpec((_BR, H), lambda i: (i, 0)),
    out_shape=jax.ShapeDtypeStruct((NP, H), jnp.float32),
)

_tc3 = pl.pallas_call(
    _tc3_body,
    grid=(_GRID,),
    in_specs=[
        pl.BlockSpec((NC, _BR, H), lambda i: (0, i, 0)),
        pl.BlockSpec((1, _BR), lambda i: (0, i)),
        pl.BlockSpec((1, H), lambda i: (0, 0)),
        pl.BlockSpec((H, C), lambda i: (0, 0)),
        pl.BlockSpec((1, C), lambda i: (0, 0)),
    ],
    out_specs=pl.BlockSpec((_BR, C), lambda i: (i, 0)),
    out_shape=jax.ShapeDtypeStruct((N, C), jnp.float32),
)


def kernel(x, edge_index, W1, b1, W2, b2, Wc, bc):
    e4 = edge_index.reshape(2, NW, NB, B)

    degp = _deg_kernel(e4)
    z1, invd = _tc1(x, W1, degp)
    p1 = _agg_kernel(z1, e4)
    z2 = _tc2(p1, invd, b1.reshape(1, H), W2)
    p2 = _agg_kernel(z2, e4)
    return _tc3(p2, invd, b2.reshape(1, H), Wc, bc.reshape(1, C))


# speedup vs baseline: 1.1059x; 1.0217x over previous
import functools

import jax
import jax.numpy as jnp
from jax import lax
from jax.experimental import pallas as pl
from jax.experimental.pallas import tpu as pltpu
import jax.experimental.pallas.tpu_sc as plsc

N = 10000
E = 320000
D = 128
H = 128
C = 70

NC = 2
NS = 16
NW = NC * NS
EPW = E // NW
B = 125
NB = EPW // B
CH = NB // 2
NP = 10240
RPT = NP // NS

_mesh = plsc.VectorSubcoreMesh(core_axis_name="c", subcore_axis_name="s",
                               num_cores=NC, num_subcores=NS)


@functools.partial(
    pl.kernel,
    out_type=jax.ShapeDtypeStruct((NC, NP), jnp.float32),
    mesh=_mesh,
    scratch_types=[
        pltpu.VMEM((NB, B), jnp.int32),
        pltpu.VMEM((128,), jnp.float32),
        pltpu.VMEM((RPT,), jnp.float32),
        pltpu.VMEM_SHARED((NP,), jnp.float32),
    ],
)
def _deg_kernel(e4_hbm, out_hbm, dst_v, ones_v, zb_v, deg_sh):
    c = lax.axis_index("c")
    s = lax.axis_index("s")
    wid = c * NS + s

    def fill_ones(i, carry):
        ones_v[pl.ds(i * 16, 16)] = jnp.full((16,), 1.0, jnp.float32)
        return carry
    lax.fori_loop(0, 8, fill_ones, 0)

    def fill_zero(i, carry):
        zb_v[pl.ds(i * 16, 16)] = jnp.zeros((16,), jnp.float32)
        return carry
    lax.fori_loop(0, RPT // 16, fill_zero, 0)

    pltpu.sync_copy(zb_v, deg_sh.at[pl.ds(s * RPT, RPT)])
    pltpu.sync_copy(e4_hbm.at[1].at[wid], dst_v)
    plsc.subcore_barrier()

    def body(j, carry):
        pltpu.sync_copy(ones_v.at[pl.ds(0, B)],
                        deg_sh.at[dst_v.at[j]], add=True)
        return carry
    lax.fori_loop(0, NB, body, 0)

    plsc.subcore_barrier()
    pltpu.sync_copy(deg_sh.at[pl.ds(s * RPT, RPT)],
                    out_hbm.at[c].at[pl.ds(s * RPT, RPT)])


@functools.partial(
    pl.kernel,
    out_type=jax.ShapeDtypeStruct((NC, NP, D), jnp.float32),
    mesh=_mesh,
    scratch_types=[
        pltpu.VMEM((CH, B), jnp.int32),
        pltpu.VMEM((CH, B), jnp.int32),
        pltpu.VMEM((B, D), jnp.float32),
        pltpu.VMEM((B, D), jnp.float32),
        pltpu.VMEM_SHARED((NP, D), jnp.float32),
        pltpu.SemaphoreType.DMA,
        pltpu.SemaphoreType.DMA,
    ],
)
def _agg_kernel(z_hbm, e4_hbm, out_hbm,
                src_v, dst_v, buf_a, buf_b, acc_sh, sem_a, sem_b):
    c = lax.axis_index("c")
    s = lax.axis_index("s")
    wid = c * NS + s

    def zrow(i, carry):
        buf_a[i // 8, pl.ds((i % 8) * 16, 16)] = jnp.zeros((16,), jnp.float32)
        return carry
    lax.fori_loop(0, B * (D // 16), zrow, 0)
    for k in range(RPT // B):
        pltpu.sync_copy(buf_a, acc_sh.at[pl.ds(s * RPT + k * B, B)])
    if RPT % B:
        pltpu.sync_copy(buf_a.at[pl.ds(0, RPT % B)],
                        acc_sh.at[pl.ds(s * RPT + (RPT // B) * B, RPT % B)])

    def load_chunk(ci):
        pltpu.sync_copy(e4_hbm.at[0].at[wid].at[pl.ds(ci * CH, CH)], src_v)
        pltpu.sync_copy(e4_hbm.at[1].at[wid].at[pl.ds(ci * CH, CH)], dst_v)

    load_chunk(0)
    plsc.subcore_barrier()

    def g_start(j, buf, sem):
        pltpu.async_copy(z_hbm.at[src_v.at[j]], buf, sem)

    def g_wait(j, buf, sem):
        pltpu.make_async_copy(z_hbm.at[src_v.at[j]], buf, sem).wait()

    def scat(j, buf):
        pltpu.sync_copy(buf, acc_sh.at[dst_v.at[j]], add=True)

    def run_pipeline(count):
        g_start(0, buf_a, sem_a)
        nloop = (count - 1) // 2 if count % 2 else (count - 2) // 2

        def body(jj, carry):
            j0 = jj * 2
            g_start(j0 + 1, buf_b, sem_b)
            g_wait(j0, buf_a, sem_a)
            scat(j0, buf_a)
            g_start(j0 + 2, buf_a, sem_a)
            g_wait(j0 + 1, buf_b, sem_b)
            scat(j0 + 1, buf_b)
            return carry
        lax.fori_loop(0, nloop, body, 0)

        if count % 2:
            g_wait(count - 1, buf_a, sem_a)
            scat(count - 1, buf_a)
        else:
            g_start(count - 1, buf_b, sem_b)
            g_wait(count - 2, buf_a, sem_a)
            scat(count - 2, buf_a)
            g_wait(count - 1, buf_b, sem_b)
            scat(count - 1, buf_b)

    run_pipeline(CH)
    load_chunk(1)
    run_pipeline(NB - CH)

    plsc.subcore_barrier()
    pltpu.sync_copy(acc_sh.at[pl.ds(s * RPT, RPT)],
                    out_hbm.at[c].at[pl.ds(s * RPT, RPT)])


_BR = 5120
_GRID = NP // _BR


def _tc1_body(x_ref, w1_ref, dp_ref, z1_ref, invd_ref):
    invd_row = lax.rsqrt(dp_ref[0:1] + dp_ref[1:2] + 1.0)
    invd_col = invd_row.reshape(_BR, 1)
    y = jnp.dot(x_ref[...], w1_ref[...], preferred_element_type=jnp.float32)
    z1_ref[...] = y * invd_col
    invd_ref[...] = invd_row


def _tc2_body(p_ref, invd_ref, b1_ref, w2_ref, z2_ref):
    invd_col = invd_ref[...].reshape(_BR, 1)
    h1 = jnp.maximum((p_ref[0] + p_ref[1]) * invd_col + b1_ref[...], 0.0)
    z2_ref[...] = jnp.dot(h1, w2_ref[...], preferred_element_type=jnp.float32) * invd_col


def _tc3_body(p_ref, invd_ref, b2_ref, wc_ref, bc_ref, out_ref):
    invd_col = invd_ref[...].reshape(_BR, 1)
    h2 = jnp.maximum((p_ref[0] + p_ref[1]) * invd_col + b2_ref[...], 0.0)
    out_ref[...] = jnp.dot(h2, wc_ref[...], preferred_element_type=jnp.float32) + bc_ref[...]


_tc1 = pl.pallas_call(
    _tc1_body,
    grid=(_GRID,),
    in_specs=[
        pl.BlockSpec((_BR, D), lambda i: (i, 0)),
        pl.BlockSpec((D, H), lambda i: (0, 0)),
        pl.BlockSpec((NC, _BR), lambda i: (0, i)),
    ],
    out_specs=[
        pl.BlockSpec((_BR, H), lambda i: (i, 0)),
        pl.BlockSpec((1, _BR), lambda i: (0, i)),
    ],
    out_shape=[
        jax.ShapeDtypeStruct((NP, H), jnp.float32),
        jax.ShapeDtypeStruct((1, NP), jnp.float32),
    ],
)

_tc2 = pl.pallas_call(
    _tc2_body,
    grid=(_GRID,),
    in_specs=[
        pl.BlockSpec((NC, _BR, H), lambda i: (0, i, 0)),
        pl.BlockSpec((1, _BR), lambda i: (0, i)),
        pl.BlockSpec((1, H), lambda i: (0, 0)),
        pl.BlockSpec((H, H), lambda i: (0, 0)),
    ],
    out_specs=pl.BlockSpec((_BR, H), lambda i: (i, 0)),
    out_shape=jax.ShapeDtypeStruct((NP, H), jnp.float32),
)

_tc3 = pl.pallas_call(
    _tc3_body,
    grid=(_GRID,),
    in_specs=[
        pl.BlockSpec((NC, _BR, H), lambda i: (0, i, 0)),
        pl.BlockSpec((1, _BR), lambda i: (0, i)),
        pl.BlockSpec((1, H), lambda i: (0, 0)),
        pl.BlockSpec((H, C), lambda i: (0, 0)),
        pl.BlockSpec((1, C), lambda i: (0, 0)),
    ],
    out_specs=pl.BlockSpec((_BR, C), lambda i: (i, 0)),
    out_shape=jax.ShapeDtypeStruct((N, C), jnp.float32),
)


def kernel(x, edge_index, W1, b1, W2, b2, Wc, bc):
    e4 = edge_index.reshape(2, NW, NB, B)

    degp = _deg_kernel(e4)
    z1, invd = _tc1(x, W1, degp)
    p1 = _agg_kernel(z1, e4)
    z2 = _tc2(p1, invd, b1.reshape(1, H), W2)
    p2 = _agg_kernel(z2, e4)
    return _tc3(p2, invd, b2.reshape(1, H), Wc, bc.reshape(1, C))
